# baseline (speedup 1.0000x reference)
import functools

import jax
import jax.numpy as jnp
from jax import lax
from jax.experimental import pallas as pl
from jax.experimental.pallas import tpu as pltpu
from jax.experimental.pallas import tpu_sc as plsc

_EPS = 1e-12

_NC = 2
_NS = 16
_NW = _NC * _NS

_K = 4
_T = 80
_BS = 16


def _sc_gather_sum(dna, ideas, idxd3, idxi3, n_tokens, chunks, h, t):
    mesh = plsc.VectorSubcoreMesh(core_axis_name="c", subcore_axis_name="s")

    @functools.partial(
        pl.kernel,
        mesh=mesh,
        out_type=jax.ShapeDtypeStruct((n_tokens, h), jnp.float32),
        scratch_types=[
            pltpu.VMEM((chunks, t), jnp.int32),
            pltpu.VMEM((chunks, t), jnp.int32),
            pltpu.VMEM((t, h), jnp.float32),
            pltpu.VMEM((t, h), jnp.float32),
            pltpu.VMEM((t, h), jnp.float32),
            pltpu.SemaphoreType.DMA,
            pltpu.SemaphoreType.DMA,
            pltpu.SemaphoreType.DMA,
            pltpu.SemaphoreType.DMA,
            pltpu.SemaphoreType.DMA,
            pltpu.SemaphoreType.DMA,
            pltpu.SemaphoreType.DMA,
            pltpu.SemaphoreType.DMA,
            pltpu.SemaphoreType.DMA,
        ],
    )
    def k(dna_h, ideas_h, idxd_h, idxi_h, out_h, idxd_v, idxi_v,
          rows0, rows1, rows2,
          semd0, sema0, semw0, semd1, sema1, semw1, semd2, sema2, semw2):
        wid = lax.axis_index("s") * _NC + lax.axis_index("c")
        pltpu.sync_copy(idxd_h.at[wid], idxd_v)
        pltpu.sync_copy(idxi_h.at[wid], idxi_v)

        bufs = ((rows0, semd0, sema0, semw0),
                (rows1, semd1, sema1, semw1),
                (rows2, semd2, sema2, semw2))

        def start_dna(c, b):
            rows, semd, _, _ = bufs[b]
            pltpu.async_copy(dna_h.at[idxd_v.at[c]], rows, semd)

        def start_add(c, b):
            rows, semd, sema, _ = bufs[b]
            pltpu.make_async_copy(dna_h.at[idxd_v.at[c]], rows, semd).wait()
            pltpu.async_copy(ideas_h.at[idxi_v.at[c]], rows, sema, add=True)

        def write_out(c, b):
            rows, _, sema, semw = bufs[b]
            pltpu.make_async_copy(ideas_h.at[idxi_v.at[c]], rows, sema).wait()
            base = (wid * chunks + c) * t
            pltpu.async_copy(rows, out_h.at[pl.ds(base, t)], semw)

        def wait_write(c, b):
            rows, _, _, semw = bufs[b]
            base = (wid * chunks + c) * t
            pltpu.make_async_copy(rows, out_h.at[pl.ds(base, t)], semw).wait()

        start_dna(0, 0)
        start_dna(1, 1)
        start_add(0, 0)

        def triple(g, carry):
            for b in range(3):
                c = 3 * g + b

                @pl.when(c + 2 < chunks)
                def _():
                    @pl.when(c >= 1)
                    def _():
                        wait_write(c - 1, (b + 2) % 3)
                    start_dna(c + 2, (b + 2) % 3)

                @pl.when(c + 1 < chunks)
                def _():
                    start_add(c + 1, (b + 1) % 3)

                @pl.when(c < chunks)
                def _():
                    write_out(c, b)
            return carry

        lax.fori_loop(0, (chunks + 2) // 3, triple, 0)
        wait_write(chunks - 3, (chunks - 3) % 3)
        wait_write(chunks - 2, (chunks - 2) % 3)
        wait_write(chunks - 1, (chunks - 1) % 3)

    return k(dna, ideas, idxd3, idxi3)


def _tc_ln_body(has_alias, x_ref, idd_ref, idi_ref, pos_ref, wd0_ref, wi0_ref,
                g_ref, b_ref, *rest):
    o_ref = rest[-1]
    bs_, s_, h = x_ref.shape
    x = x_ref[...]
    md = (idd_ref[...] == 0).astype(jnp.float32)[..., None]
    mi = (idi_ref[...] == 0).astype(jnp.float32)[..., None]
    x = (x
         - md * wd0_ref[0][None, None, :]
         - mi * wi0_ref[0][None, None, :]
         + pos_ref[...][None, :, :])
    x2 = x.reshape(bs_ * s_, h)
    jmat = jnp.full((h, h), 1.0 / h, dtype=jnp.float32)
    m = jax.lax.dot_general(x2, jmat, (((1,), (0,)), ((), ())),
                            preferred_element_type=jnp.float32)
    q = jax.lax.dot_general(x2 * x2, jmat, (((1,), (0,)), ((), ())),
                            preferred_element_type=jnp.float32)
    r = lax.rsqrt(q - m * m + _EPS)
    o = (x2 - m) * r * g_ref[0][None, :] + b_ref[0][None, :]
    o_ref[...] = o.reshape(bs_, s_, h)


def _tc_ln_slice(row0, brows, sums_k, ids_d, ids_i, pos, wd0, wi0, gamma2,
                 beta2, prev_buf):
    b, s = ids_d.shape
    h = pos.shape[-1]
    steps = brows // _BS
    blk0 = row0 // _BS
    sums3 = sums_k.reshape(brows, s, h)

    in_specs = [
        pl.BlockSpec((_BS, s, h), lambda i: (i, 0, 0)),
        pl.BlockSpec((_BS, s), lambda i: (blk0 + i, 0)),
        pl.BlockSpec((_BS, s), lambda i: (blk0 + i, 0)),
        pl.BlockSpec((s, h), lambda i: (0, 0)),
        pl.BlockSpec((1, h), lambda i: (0, 0)),
        pl.BlockSpec((1, h), lambda i: (0, 0)),
        pl.BlockSpec((1, h), lambda i: (0, 0)),
        pl.BlockSpec((1, h), lambda i: (0, 0)),
    ]
    args = [sums3, ids_d, ids_i, pos, wd0, wi0, gamma2, beta2]
    aliases = {}
    if prev_buf is not None:
        in_specs.append(pl.BlockSpec(memory_space=pl.ANY))
        args.append(prev_buf)
        aliases = {8: 0}
    return pl.pallas_call(
        functools.partial(_tc_ln_body, prev_buf is not None),
        grid=(steps,),
        in_specs=in_specs,
        out_specs=pl.BlockSpec((_BS, s, h), lambda i: (blk0 + i, 0, 0)),
        out_shape=jax.ShapeDtypeStruct((b, s, h), jnp.float32),
        input_output_aliases=aliases,
    )(*args)


_SLICES = (256, 256, 256, 256)


def kernel(input_ids_dna, input_ids_ideas, W_dna, W_ideas, W_pos, gamma, beta):
    b, s = input_ids_dna.shape
    v, h = W_dna.shape
    n_tokens = b * s

    idd_flat = input_ids_dna.reshape(n_tokens)
    idi_flat = input_ids_ideas.reshape(n_tokens)

    pos = W_pos[:s]
    wd0 = W_dna[0:1]
    wi0 = W_ideas[0:1]
    gamma2 = gamma.reshape(1, h)
    beta2 = beta.reshape(1, h)

    sums = []
    row0 = 0
    for brows in _SLICES:
        n_slice = brows * s
        chunks = n_slice // (_NW * _T)
        t0 = row0 * s
        idxd = lax.slice(idd_flat, (t0,), (t0 + n_slice,))
        idxi = lax.slice(idi_flat, (t0,), (t0 + n_slice,))
        sums.append(_sc_gather_sum(
            W_dna, W_ideas,
            idxd.reshape(_NW, chunks, _T), idxi.reshape(_NW, chunks, _T),
            n_slice, chunks, h, _T))
        row0 += brows

    buf = None
    row0 = 0
    for brows, sums_k in zip(_SLICES, sums):
        buf = _tc_ln_slice(row0, brows, sums_k, input_ids_dna, input_ids_ideas,
                           pos, wd0, wi0, gamma2, beta2, buf)
        row0 += brows
    return buf


# speedup vs baseline: 1.0074x; 1.0074x over previous
import functools

import jax
import jax.numpy as jnp
from jax import lax
from jax.experimental import pallas as pl
from jax.experimental.pallas import tpu as pltpu
from jax.experimental.pallas import tpu_sc as plsc

_EPS = 1e-12

_NC = 2
_NS = 16
_NW = _NC * _NS

_T = 80
_BS = 32


def _sc_gather_sum(dna, ideas, idxd3, idxi3, n_tokens, chunks, h, t):
    mesh = plsc.VectorSubcoreMesh(core_axis_name="c", subcore_axis_name="s")

    @functools.partial(
        pl.kernel,
        mesh=mesh,
        out_type=jax.ShapeDtypeStruct((n_tokens, h), jnp.float32),
        scratch_types=[
            pltpu.VMEM((chunks, t), jnp.int32),
            pltpu.VMEM((chunks, t), jnp.int32),
            pltpu.VMEM((t, h), jnp.float32),
            pltpu.VMEM((t, h), jnp.float32),
            pltpu.VMEM((t, h), jnp.float32),
            pltpu.SemaphoreType.DMA,
            pltpu.SemaphoreType.DMA,
            pltpu.SemaphoreType.DMA,
            pltpu.SemaphoreType.DMA,
            pltpu.SemaphoreType.DMA,
            pltpu.SemaphoreType.DMA,
            pltpu.SemaphoreType.DMA,
            pltpu.SemaphoreType.DMA,
            pltpu.SemaphoreType.DMA,
        ],
    )
    def k(dna_h, ideas_h, idxd_h, idxi_h, out_h, idxd_v, idxi_v,
          rows0, rows1, rows2,
          semd0, sema0, semw0, semd1, sema1, semw1, semd2, sema2, semw2):
        wid = lax.axis_index("s") * _NC + lax.axis_index("c")
        pltpu.sync_copy(idxd_h.at[wid], idxd_v)
        pltpu.sync_copy(idxi_h.at[wid], idxi_v)

        bufs = ((rows0, semd0, sema0, semw0),
                (rows1, semd1, sema1, semw1),
                (rows2, semd2, sema2, semw2))

        def start_dna(c, b):
            rows, semd, _, _ = bufs[b]
            pltpu.async_copy(dna_h.at[idxd_v.at[c]], rows, semd)

        def start_add(c, b):
            rows, semd, sema, _ = bufs[b]
            pltpu.make_async_copy(dna_h.at[idxd_v.at[c]], rows, semd).wait()
            pltpu.async_copy(ideas_h.at[idxi_v.at[c]], rows, sema, add=True)

        def write_out(c, b):
            rows, _, sema, semw = bufs[b]
            pltpu.make_async_copy(ideas_h.at[idxi_v.at[c]], rows, sema).wait()
            base = (wid * chunks + c) * t
            pltpu.async_copy(rows, out_h.at[pl.ds(base, t)], semw)

        def wait_write(c, b):
            rows, _, _, semw = bufs[b]
            base = (wid * chunks + c) * t
            pltpu.make_async_copy(rows, out_h.at[pl.ds(base, t)], semw).wait()

        start_dna(0, 0)
        start_dna(1, 1)
        start_add(0, 0)

        def triple(g, carry):
            for b in range(3):
                c = 3 * g + b

                @pl.when(c + 2 < chunks)
                def _():
                    @pl.when(c >= 1)
                    def _():
                        wait_write(c - 1, (b + 2) % 3)
                    start_dna(c + 2, (b + 2) % 3)

                @pl.when(c + 1 < chunks)
                def _():
                    start_add(c + 1, (b + 1) % 3)

                @pl.when(c < chunks)
                def _():
                    write_out(c, b)
            return carry

        lax.fori_loop(0, (chunks + 2) // 3, triple, 0)
        wait_write(chunks - 3, (chunks - 3) % 3)
        wait_write(chunks - 2, (chunks - 2) % 3)
        wait_write(chunks - 1, (chunks - 1) % 3)

    return k(dna, ideas, idxd3, idxi3)


def _tc_ln_body(has_alias, x_ref, idd_ref, idi_ref, pos_ref, wd0_ref, wi0_ref,
                g_ref, b_ref, *rest):
    o_ref = rest[-1]
    bs_, s_, h = x_ref.shape
    x = x_ref[...]
    md = (idd_ref[...] == 0).astype(jnp.float32)[..., None]
    mi = (idi_ref[...] == 0).astype(jnp.float32)[..., None]
    x = (x
         - md * wd0_ref[0][None, None, :]
         - mi * wi0_ref[0][None, None, :]
         + pos_ref[...][None, :, :])
    jmat = jnp.full((h, h), 1.0 / h, dtype=jnp.float32)
    dnums = (((2,), (0,)), ((), ()))
    m = jax.lax.dot_general(x, jmat, dnums,
                            preferred_element_type=jnp.float32)
    q = jax.lax.dot_general(x * x, jmat, dnums,
                            preferred_element_type=jnp.float32)
    r = lax.rsqrt(q - m * m + _EPS)
    o_ref[...] = ((x - m) * r * g_ref[0][None, None, :]
                  + b_ref[0][None, None, :])


def _tc_ln_slice(row0, brows, sums_k, ids_d, ids_i, pos, wd0, wi0, gamma2,
                 beta2, prev_buf):
    b, s = ids_d.shape
    h = pos.shape[-1]
    steps = brows // _BS
    blk0 = row0 // _BS
    sums3 = sums_k.reshape(brows, s, h)

    in_specs = [
        pl.BlockSpec((_BS, s, h), lambda i: (i, 0, 0)),
        pl.BlockSpec((_BS, s), lambda i: (blk0 + i, 0)),
        pl.BlockSpec((_BS, s), lambda i: (blk0 + i, 0)),
        pl.BlockSpec((s, h), lambda i: (0, 0)),
        pl.BlockSpec((1, h), lambda i: (0, 0)),
        pl.BlockSpec((1, h), lambda i: (0, 0)),
        pl.BlockSpec((1, h), lambda i: (0, 0)),
        pl.BlockSpec((1, h), lambda i: (0, 0)),
    ]
    args = [sums3, ids_d, ids_i, pos, wd0, wi0, gamma2, beta2]
    aliases = {}
    if prev_buf is not None:
        in_specs.append(pl.BlockSpec(memory_space=pl.ANY))
        args.append(prev_buf)
        aliases = {8: 0}
    return pl.pallas_call(
        functools.partial(_tc_ln_body, prev_buf is not None),
        grid=(steps,),
        in_specs=in_specs,
        out_specs=pl.BlockSpec((_BS, s, h), lambda i: (blk0 + i, 0, 0)),
        out_shape=jax.ShapeDtypeStruct((b, s, h), jnp.float32),
        input_output_aliases=aliases,
    )(*args)


_SLICES = (256, 256, 256, 256)


def kernel(input_ids_dna, input_ids_ideas, W_dna, W_ideas, W_pos, gamma, beta):
    b, s = input_ids_dna.shape
    v, h = W_dna.shape
    n_tokens = b * s

    idd_flat = input_ids_dna.reshape(n_tokens)
    idi_flat = input_ids_ideas.reshape(n_tokens)

    pos = W_pos[:s]
    wd0 = W_dna[0:1]
    wi0 = W_ideas[0:1]
    gamma2 = gamma.reshape(1, h)
    beta2 = beta.reshape(1, h)

    sums = []
    row0 = 0
    for brows in _SLICES:
        n_slice = brows * s
        chunks = n_slice // (_NW * _T)
        t0 = row0 * s
        idxd = lax.slice(idd_flat, (t0,), (t0 + n_slice,))
        idxi = lax.slice(idi_flat, (t0,), (t0 + n_slice,))
        sums.append(_sc_gather_sum(
            W_dna, W_ideas,
            idxd.reshape(_NW, chunks, _T), idxi.reshape(_NW, chunks, _T),
            n_slice, chunks, h, _T))
        row0 += brows

    buf = None
    row0 = 0
    for brows, sums_k in zip(_SLICES, sums):
        buf = _tc_ln_slice(row0, brows, sums_k, input_ids_dna, input_ids_ideas,
                           pos, wd0, wi0, gamma2, beta2, buf)
        row0 += brows
    return buf


# speedup vs baseline: 1.0140x; 1.0066x over previous
import functools

import jax
import jax.numpy as jnp
from jax import lax
from jax.experimental import pallas as pl
from jax.experimental.pallas import tpu as pltpu
from jax.experimental.pallas import tpu_sc as plsc

_EPS = 1e-12

_NC = 2
_NS = 16
_NW = _NC * _NS

_T = 80
_BS = 64


def _sc_gather_sum(dna, ideas, idxd3, idxi3, n_tokens, chunks, h, t):
    mesh = plsc.VectorSubcoreMesh(core_axis_name="c", subcore_axis_name="s")

    @functools.partial(
        pl.kernel,
        mesh=mesh,
        out_type=jax.ShapeDtypeStruct((n_tokens, h), jnp.float32),
        scratch_types=[
            pltpu.VMEM((chunks, t), jnp.int32),
            pltpu.VMEM((chunks, t), jnp.int32),
            pltpu.VMEM((t, h), jnp.float32),
            pltpu.VMEM((t, h), jnp.float32),
            pltpu.VMEM((t, h), jnp.float32),
            pltpu.SemaphoreType.DMA,
            pltpu.SemaphoreType.DMA,
            pltpu.SemaphoreType.DMA,
            pltpu.SemaphoreType.DMA,
            pltpu.SemaphoreType.DMA,
            pltpu.SemaphoreType.DMA,
            pltpu.SemaphoreType.DMA,
            pltpu.SemaphoreType.DMA,
            pltpu.SemaphoreType.DMA,
        ],
    )
    def k(dna_h, ideas_h, idxd_h, idxi_h, out_h, idxd_v, idxi_v,
          rows0, rows1, rows2,
          semd0, sema0, semw0, semd1, sema1, semw1, semd2, sema2, semw2):
        wid = lax.axis_index("s") * _NC + lax.axis_index("c")
        pltpu.sync_copy(idxd_h.at[wid], idxd_v)
        pltpu.sync_copy(idxi_h.at[wid], idxi_v)

        bufs = ((rows0, semd0, sema0, semw0),
                (rows1, semd1, sema1, semw1),
                (rows2, semd2, sema2, semw2))

        def start_dna(c, b):
            rows, semd, _, _ = bufs[b]
            pltpu.async_copy(dna_h.at[idxd_v.at[c]], rows, semd)

        def start_add(c, b):
            rows, semd, sema, _ = bufs[b]
            pltpu.make_async_copy(dna_h.at[idxd_v.at[c]], rows, semd).wait()
            pltpu.async_copy(ideas_h.at[idxi_v.at[c]], rows, sema, add=True)

        def write_out(c, b):
            rows, _, sema, semw = bufs[b]
            pltpu.make_async_copy(ideas_h.at[idxi_v.at[c]], rows, sema).wait()
            base = (wid * chunks + c) * t
            pltpu.async_copy(rows, out_h.at[pl.ds(base, t)], semw)

        def wait_write(c, b):
            rows, _, _, semw = bufs[b]
            base = (wid * chunks + c) * t
            pltpu.make_async_copy(rows, out_h.at[pl.ds(base, t)], semw).wait()

        start_dna(0, 0)
        start_dna(1, 1)
        start_add(0, 0)

        def triple(g, carry):
            for b in range(3):
                c = 3 * g + b

                @pl.when(c + 2 < chunks)
                def _():
                    @pl.when(c >= 1)
                    def _():
                        wait_write(c - 1, (b + 2) % 3)
                    start_dna(c + 2, (b + 2) % 3)

                @pl.when(c + 1 < chunks)
                def _():
                    start_add(c + 1, (b + 1) % 3)

                @pl.when(c < chunks)
                def _():
                    write_out(c, b)
            return carry

        lax.fori_loop(0, (chunks + 2) // 3, triple, 0)
        wait_write(chunks - 3, (chunks - 3) % 3)
        wait_write(chunks - 2, (chunks - 2) % 3)
        wait_write(chunks - 1, (chunks - 1) % 3)

    return k(dna, ideas, idxd3, idxi3)


def _tc_ln_body(has_alias, x_ref, idd_ref, idi_ref, pos_ref, wd0_ref, wi0_ref,
                g_ref, b_ref, *rest):
    o_ref = rest[-1]
    bs_, s_, h = x_ref.shape
    x = x_ref[...]
    md = (idd_ref[...] == 0).astype(jnp.float32)[..., None]
    mi = (idi_ref[...] == 0).astype(jnp.float32)[..., None]
    x = (x
         - md * wd0_ref[0][None, None, :]
         - mi * wi0_ref[0][None, None, :]
         + pos_ref[...][None, :, :])
    jmat = jnp.full((h, h), 1.0 / h, dtype=jnp.float32)
    dnums = (((2,), (0,)), ((), ()))
    m = jax.lax.dot_general(x, jmat, dnums,
                            preferred_element_type=jnp.float32)
    q = jax.lax.dot_general(x * x, jmat, dnums,
                            preferred_element_type=jnp.float32)
    r = lax.rsqrt(q - m * m + _EPS)
    o_ref[...] = ((x - m) * r * g_ref[0][None, None, :]
                  + b_ref[0][None, None, :])


def _tc_ln_slice(row0, brows, sums_k, ids_d, ids_i, pos, wd0, wi0, gamma2,
                 beta2, prev_buf):
    b, s = ids_d.shape
    h = pos.shape[-1]
    steps = brows // _BS
    blk0 = row0 // _BS
    sums3 = sums_k.reshape(brows, s, h)

    in_specs = [
        pl.BlockSpec((_BS, s, h), lambda i: (i, 0, 0)),
        pl.BlockSpec((_BS, s), lambda i: (blk0 + i, 0)),
        pl.BlockSpec((_BS, s), lambda i: (blk0 + i, 0)),
        pl.BlockSpec((s, h), lambda i: (0, 0)),
        pl.BlockSpec((1, h), lambda i: (0, 0)),
        pl.BlockSpec((1, h), lambda i: (0, 0)),
        pl.BlockSpec((1, h), lambda i: (0, 0)),
        pl.BlockSpec((1, h), lambda i: (0, 0)),
    ]
    args = [sums3, ids_d, ids_i, pos, wd0, wi0, gamma2, beta2]
    aliases = {}
    if prev_buf is not None:
        in_specs.append(pl.BlockSpec(memory_space=pl.ANY))
        args.append(prev_buf)
        aliases = {8: 0}
    return pl.pallas_call(
        functools.partial(_tc_ln_body, prev_buf is not None),
        grid=(steps,),
        in_specs=in_specs,
        out_specs=pl.BlockSpec((_BS, s, h), lambda i: (blk0 + i, 0, 0)),
        out_shape=jax.ShapeDtypeStruct((b, s, h), jnp.float32),
        input_output_aliases=aliases,
    )(*args)


_SLICES = (256, 256, 256, 256)


def kernel(input_ids_dna, input_ids_ideas, W_dna, W_ideas, W_pos, gamma, beta):
    b, s = input_ids_dna.shape
    v, h = W_dna.shape
    n_tokens = b * s

    idd_flat = input_ids_dna.reshape(n_tokens)
    idi_flat = input_ids_ideas.reshape(n_tokens)

    pos = W_pos[:s]
    wd0 = W_dna[0:1]
    wi0 = W_ideas[0:1]
    gamma2 = gamma.reshape(1, h)
    beta2 = beta.reshape(1, h)

    sums = []
    row0 = 0
    for brows in _SLICES:
        n_slice = brows * s
        chunks = n_slice // (_NW * _T)
        t0 = row0 * s
        idxd = lax.slice(idd_flat, (t0,), (t0 + n_slice,))
        idxi = lax.slice(idi_flat, (t0,), (t0 + n_slice,))
        sums.append(_sc_gather_sum(
            W_dna, W_ideas,
            idxd.reshape(_NW, chunks, _T), idxi.reshape(_NW, chunks, _T),
            n_slice, chunks, h, _T))
        row0 += brows

    buf = None
    row0 = 0
    for brows, sums_k in zip(_SLICES, sums):
        buf = _tc_ln_slice(row0, brows, sums_k, input_ids_dna, input_ids_ideas,
                           pos, wd0, wi0, gamma2, beta2, buf)
        row0 += brows
    return buf
